# baseline (speedup 1.0000x reference)
import functools

import jax
import jax.numpy as jnp
from jax import lax
from jax.experimental import pallas as pl
from jax.experimental.pallas import tpu as pltpu
from jax.experimental.pallas import tpu_sc as plsc

_M, _H, _K = 160000, 128, 6
_DEPTH = 2
_NLOG2E = -1.0
_HW = _H // 2

_BM = 2000
_NBLK = _M // _BM

_NC, _NS = 2, 16
_NW = _NC * _NS
_ROWS_W = _M // _NW
_CH = 20
_GI = _CH * _K
_NCH = _ROWS_W // _CH
_MA, _MB = 96000, 64000


def _dotT(x, w):
    return lax.dot_general(x.astype(jnp.bfloat16), w.astype(jnp.bfloat16),
                           (((1,), (1,)), ((), ())),
                           preferred_element_type=jnp.float32)


def _exp_neg(x):
    return jnp.exp(jnp.minimum(-x, 20.0))


def _pack_words(x):
    rn = lambda v: v.astype(jnp.bfloat16).astype(jnp.float32)
    return pltpu.pack_elementwise(
        [rn(x[:, :_HW]), rn(x[:, _HW:])], packed_dtype=jnp.bfloat16)


def _unpack_words(x):
    un = lambda i: pltpu.unpack_elementwise(
        x, index=i, packed_dtype=jnp.bfloat16, unpacked_dtype=jnp.float32)
    return jnp.concatenate([un(0), un(1)], axis=1)


def _pre_body(lf_ref, msg_ref, wz_ref, wr_ref, ur_ref, urb_ref, wh_ref,
              r1_ref, lfzh_ref, mcat_ref):
    lf = lf_ref[...]
    msg = msg_ref[...]
    wz = wz_ref[...]
    wh = wh_ref[...]
    r1_ref[...] = _pack_words(_exp_neg(_dotT(lf, wr_ref[...])))
    lfzh_ref[...] = jnp.concatenate(
        [_pack_words(_dotT(lf, wz[:, :_H])),
         _pack_words(_dotT(lf, wh[:, :_H]))], axis=1)
    mcat_ref[...] = jnp.concatenate(
        [_pack_words(msg),
         _pack_words(_exp_neg(_dotT(msg, ur_ref[...]) + urb_ref[...]))],
        axis=1)


def _tc_pre(lf, msgs, Wz_w, Wr_w, Ur_w, Ur_b2, Wh_w):
    row = lambda i: (i, 0)
    rep = lambda i: (0, 0)
    return pl.pallas_call(
        _pre_body,
        grid=(_NBLK,),
        in_specs=[
            pl.BlockSpec((_BM, _H), row),
            pl.BlockSpec((_BM, _H), row),
            pl.BlockSpec((_H, 2 * _H), rep),
            pl.BlockSpec((_H, _H), rep),
            pl.BlockSpec((_H, _H), rep),
            pl.BlockSpec((1, _H), rep),
            pl.BlockSpec((_H, 2 * _H), rep),
        ],
        out_specs=[
            pl.BlockSpec((_BM, _HW), row),
            pl.BlockSpec((_BM, _H), row),
            pl.BlockSpec((_BM, _H), row),
        ],
        out_shape=[
            jax.ShapeDtypeStruct((_M, _HW), jnp.int32),
            jax.ShapeDtypeStruct((_M, _H), jnp.int32),
            jax.ShapeDtypeStruct((_M, _H), jnp.int32),
        ],
    )(lf, msgs, Wz_w, Wr_w, Ur_w, Ur_b2, Wh_w)


def _post_body(with_mu, mask0, sums_ref, lfzh_ref, wz_ref, wh_ref, ur_ref,
               wzb_ref, whb_ref, urb_ref, out_ref):
    sums = sums_ref[...]
    sm = _unpack_words(sums[:, :_HW])
    sg = _unpack_words(sums[:, _HW:])
    lfzh = lfzh_ref[...]
    z = jax.nn.sigmoid(_unpack_words(lfzh[:, :_HW])
                       + _dotT(sm, wz_ref[...][:, _H:]) + wzb_ref[...])
    pre = jnp.tanh(_unpack_words(lfzh[:, _HW:])
                   + _dotT(sg, wh_ref[...][:, _H:]) + whb_ref[...])
    mnew = (1.0 - z) * sm + z * pre
    if mask0:
        rows = lax.broadcasted_iota(jnp.int32, (_BM, 1), 0)
        mnew = jnp.where((pl.program_id(0) == 0) & (rows == 0), 0.0, mnew)
    if with_mu:
        out_ref[...] = jnp.concatenate(
            [_pack_words(mnew),
             _pack_words(_exp_neg(_dotT(mnew, ur_ref[...])
                                  + urb_ref[...]))], axis=1)
    else:
        out_ref[...] = mnew


def _tc_post(sums, lfzh, Wz_w, Wh_w, Ur_w, Wz_b2, Wh_b2, Ur_b2, with_mu,
             mask0):
    row = lambda i: (i, 0)
    rep = lambda i: (0, 0)
    out_w = _H
    rows_w = sums.shape[0]
    return pl.pallas_call(
        functools.partial(_post_body, with_mu, mask0),
        grid=(rows_w // _BM,),
        in_specs=[
            pl.BlockSpec((_BM, _H), row),
            pl.BlockSpec((_BM, _H), row),
            pl.BlockSpec((_H, 2 * _H), rep),
            pl.BlockSpec((_H, 2 * _H), rep),
            pl.BlockSpec((_H, _H), rep),
            pl.BlockSpec((1, _H), rep),
            pl.BlockSpec((1, _H), rep),
            pl.BlockSpec((1, _H), rep),
        ],
        out_specs=pl.BlockSpec((_BM, out_w), row),
        out_shape=jax.ShapeDtypeStruct(
            (rows_w, out_w), jnp.int32 if with_mu else jnp.float32),
    )(sums, lfzh, Wz_w, Wh_w, Ur_w, Wz_b2, Wh_b2, Ur_b2)


def _make_sc_body(row_base, rows_w):
  rows_pw = rows_w // _NW
  nch = rows_pw // _CH

  def _sc_body(mcat_hbm, r1_hbm, idx_hbm, out_hbm,
               idx_all, rows0, rows1, r1b0, r1b1, sums0, sums1,
               g0, g1, r0, r1s, s0, s1):
    wid = lax.axis_index("s") * _NC + lax.axis_index("c")
    wbase = row_base + wid * rows_pw
    obase = wid * rows_pw
    pltpu.sync_copy(idx_hbm.at[pl.ds(wbase * _K, rows_pw * _K)], idx_all)

    def fetch(c, rows_b, r1_b, gsem, rsem):
        pltpu.async_copy(
            r1_hbm.at[pl.ds((wbase + c * _CH) * _HW, _CH * _HW)], r1_b,
            rsem)
        pltpu.async_copy(
            mcat_hbm.at[idx_all.at[pl.ds(c * _GI, _GI)]], rows_b, gsem)

    def wait_fetch(c, rows_b, r1_b, gsem, rsem):
        pltpu.make_async_copy(
            r1_hbm.at[pl.ds((wbase + c * _CH) * _HW, _CH * _HW)], r1_b,
            rsem).wait()
        pltpu.make_async_copy(
            mcat_hbm.at[idx_all.at[pl.ds(c * _GI, _GI)]], rows_b,
            gsem).wait()

    def compute(c, rows_b, r1_b, sums_b):
        bf = jnp.bfloat16

        def rowfn(i2, carry2):
            for r in range(2):
                i = 2 * i2 + r
                i6 = i * _K
                for g in range(_H // 32):
                    sl = pl.ds(g * 16, 16)
                    slu = pl.ds(_HW + g * 16, 16)
                    r1v = plsc.bitcast(
                        r1_b[pl.ds(i * _HW + g * 16, 16)], bf)
                    mvs = [plsc.bitcast(rows_b[i6 + k, sl], bf)
                           for k in range(_K)]
                    uvs = [plsc.bitcast(rows_b[i6 + k, slu], bf)
                           for k in range(_K)]
                    gs = [mvs[k] / (1.0 + r1v * uvs[k])
                          for k in range(_K)]
                    acc_m = ((mvs[0] + mvs[1]) + (mvs[2] + mvs[3])
                             + (mvs[4] + mvs[5]))
                    acc_g = ((gs[0] + gs[1]) + (gs[2] + gs[3])
                             + (gs[4] + gs[5]))
                    base = i * _H + g * 16
                    sums_b[pl.ds(base, 16)] = plsc.bitcast(acc_m, jnp.int32)
                    sums_b[pl.ds(base + _HW, 16)] = (
                        plsc.bitcast(acc_g, jnp.int32))
            return carry2
        lax.fori_loop(0, _CH // 2, rowfn, 0)

    def store(c, sums_b, ssem):
        pltpu.async_copy(
            sums_b,
            out_hbm.at[pl.ds((obase + c * _CH) * _H, _CH * _H)],
            ssem)

    def wait_store(c, sums_b, ssem):
        pltpu.make_async_copy(
            sums_b,
            out_hbm.at[pl.ds((obase + c * _CH) * _H, _CH * _H)],
            ssem).wait()

    fetch(0, rows0, r1b0, g0, r0)
    fetch(1, rows1, r1b1, g1, r1s)

    def pair(t, carry):
        c0 = 2 * t
        c1 = c0 + 1
        wait_fetch(c0, rows0, r1b0, g0, r0)

        @pl.when(t > 0)
        def _():
            wait_store(c0 - 2, sums0, s0)

        compute(c0, rows0, r1b0, sums0)
        store(c0, sums0, s0)

        @pl.when(t < nch // 2 - 1)
        def _():
            fetch(c0 + 2, rows0, r1b0, g0, r0)

        wait_fetch(c1, rows1, r1b1, g1, r1s)

        @pl.when(t > 0)
        def _():
            wait_store(c1 - 2, sums1, s1)

        compute(c1, rows1, r1b1, sums1)
        store(c1, sums1, s1)

        @pl.when(t < nch // 2 - 1)
        def _():
            fetch(c1 + 2, rows1, r1b1, g1, r1s)

        return carry

    lax.fori_loop(0, nch // 2, pair, 0)
    wait_store(nch - 2, sums0, s0)
    wait_store(nch - 1, sums1, s1)

  return _sc_body


def _sc_gate(mcat, r1, idx, row_base, rows_w):
    fn = functools.partial(
        pl.kernel,
        mesh=plsc.VectorSubcoreMesh(core_axis_name="c", subcore_axis_name="s"),
        compiler_params=pltpu.CompilerParams(needs_layout_passes=False),
        out_type=jax.ShapeDtypeStruct((rows_w * _H,), jnp.int32),
        scratch_types=[
            pltpu.VMEM((rows_w // _NW * _K,), jnp.int32),
            pltpu.VMEM((_GI, _H), jnp.int32),
            pltpu.VMEM((_GI, _H), jnp.int32),
            pltpu.VMEM((_CH * _HW,), jnp.int32),
            pltpu.VMEM((_CH * _HW,), jnp.int32),
            pltpu.VMEM((_CH * _H,), jnp.int32),
            pltpu.VMEM((_CH * _H,), jnp.int32),
            pltpu.SemaphoreType.DMA,
            pltpu.SemaphoreType.DMA,
            pltpu.SemaphoreType.DMA,
            pltpu.SemaphoreType.DMA,
            pltpu.SemaphoreType.DMA,
            pltpu.SemaphoreType.DMA,
        ],
    )(_make_sc_body(row_base, rows_w))
    return fn(mcat, r1.reshape(-1), idx).reshape(rows_w, _H)


def kernel(messages, local_field, mess_graph, Wz_w, Wz_b, Wr_w, Ur_w, Ur_b,
           Wh_w, Wh_b):
    Wz_b2 = Wz_b.reshape(1, _H)
    Wh_b2 = Wh_b.reshape(1, _H)
    Ur_b2 = Ur_b.reshape(1, _H)
    idx = mess_graph.reshape(-1)

    r1, lfzh, mcat = _tc_pre(local_field, messages, Wz_w, Wr_w, Ur_w,
                             Ur_b2, Wh_w)
    for d in range(_DEPTH):
        with_mu = d < _DEPTH - 1
        sums_a = _sc_gate(mcat, r1, idx, 0, _MA)
        sums_b = _sc_gate(mcat, r1, idx, _MA, _MB)
        out_a = _tc_post(sums_a, lfzh[:_MA], Wz_w, Wh_w, Ur_w, Wz_b2,
                         Wh_b2, Ur_b2, with_mu, True)
        out_b = _tc_post(sums_b, lfzh[_MA:], Wz_w, Wh_w, Ur_w, Wz_b2,
                         Wh_b2, Ur_b2, with_mu, False)
        mcat = jnp.concatenate([out_a, out_b], axis=0)
    return mcat


# speedup vs baseline: 1.0321x; 1.0321x over previous
import functools

import jax
import jax.numpy as jnp
from jax import lax
from jax.experimental import pallas as pl
from jax.experimental.pallas import tpu as pltpu
from jax.experimental.pallas import tpu_sc as plsc

_M, _H, _K = 160000, 128, 6
_DEPTH = 2
_NLOG2E = -1.0
_HW = _H // 2

_BM = 2000
_NBLK = _M // _BM

_NC, _NS = 2, 16
_NW = _NC * _NS
_ROWS_W = _M // _NW
_CH = 20
_GI = _CH * _K
_NCH = _ROWS_W // _CH


def _dotT(x, w):
    return lax.dot_general(x, w, (((1,), (1,)), ((), ())),
                           preferred_element_type=jnp.float32)


def _exp_neg(x):
    return jnp.exp(jnp.minimum(-x, 20.0))


def _pack_words(x):
    rn = lambda v: v.astype(jnp.bfloat16).astype(jnp.float32)
    return pltpu.pack_elementwise(
        [rn(x[:, :_HW]), rn(x[:, _HW:])], packed_dtype=jnp.bfloat16)


def _unpack_words(x):
    un = lambda i: pltpu.unpack_elementwise(
        x, index=i, packed_dtype=jnp.bfloat16, unpacked_dtype=jnp.float32)
    return jnp.concatenate([un(0), un(1)], axis=1)


def _pre_body(lf_ref, msg_ref, wz_ref, wr_ref, ur_ref, urb_ref, wh_ref,
              r1_ref, lfzh_ref, mcat_ref):
    lf = lf_ref[...]
    msg = msg_ref[...]
    wz = wz_ref[...]
    wh = wh_ref[...]
    r1_ref[...] = _pack_words(_exp_neg(_dotT(lf, wr_ref[...])))
    lfzh_ref[...] = jnp.concatenate(
        [_dotT(lf, wz[:, :_H]), _dotT(lf, wh[:, :_H])], axis=1)
    mcat_ref[...] = jnp.concatenate(
        [_pack_words(msg),
         _pack_words(_exp_neg(_dotT(msg, ur_ref[...]) + urb_ref[...]))],
        axis=1)


def _tc_pre(lf, msgs, Wz_w, Wr_w, Ur_w, Ur_b2, Wh_w):
    row = lambda i: (i, 0)
    rep = lambda i: (0, 0)
    return pl.pallas_call(
        _pre_body,
        grid=(_NBLK,),
        in_specs=[
            pl.BlockSpec((_BM, _H), row),
            pl.BlockSpec((_BM, _H), row),
            pl.BlockSpec((_H, 2 * _H), rep),
            pl.BlockSpec((_H, _H), rep),
            pl.BlockSpec((_H, _H), rep),
            pl.BlockSpec((1, _H), rep),
            pl.BlockSpec((_H, 2 * _H), rep),
        ],
        out_specs=[
            pl.BlockSpec((_BM, _HW), row),
            pl.BlockSpec((_BM, 2 * _H), row),
            pl.BlockSpec((_BM, _H), row),
        ],
        out_shape=[
            jax.ShapeDtypeStruct((_M, _HW), jnp.int32),
            jax.ShapeDtypeStruct((_M, 2 * _H), jnp.float32),
            jax.ShapeDtypeStruct((_M, _H), jnp.int32),
        ],
    )(lf, msgs, Wz_w, Wr_w, Ur_w, Ur_b2, Wh_w)


def _post_body(with_mu, sums_ref, lfzh_ref, wz_ref, wh_ref, ur_ref,
               wzb_ref, whb_ref, urb_ref, out_ref):
    sums = sums_ref[...]
    sm = _unpack_words(sums[:, :_HW])
    sg = _unpack_words(sums[:, _HW:])
    lfzh = lfzh_ref[...]
    z = jax.nn.sigmoid(lfzh[:, :_H] + _dotT(sm, wz_ref[...][:, _H:])
                       + wzb_ref[...])
    pre = jnp.tanh(lfzh[:, _H:] + _dotT(sg, wh_ref[...][:, _H:])
                   + whb_ref[...])
    mnew = (1.0 - z) * sm + z * pre
    rows = lax.broadcasted_iota(jnp.int32, (_BM, 1), 0)
    mnew = jnp.where((pl.program_id(0) == 0) & (rows == 0), 0.0, mnew)
    if with_mu:
        out_ref[...] = jnp.concatenate(
            [_pack_words(mnew),
             _pack_words(_exp_neg(_dotT(mnew, ur_ref[...])
                                  + urb_ref[...]))], axis=1)
    else:
        out_ref[...] = mnew


def _tc_post(sums, lfzh, Wz_w, Wh_w, Ur_w, Wz_b2, Wh_b2, Ur_b2, with_mu):
    row = lambda i: (i, 0)
    rep = lambda i: (0, 0)
    out_w = _H if with_mu else _H
    return pl.pallas_call(
        functools.partial(_post_body, with_mu),
        grid=(_NBLK,),
        in_specs=[
            pl.BlockSpec((_BM, _H), row),
            pl.BlockSpec((_BM, 2 * _H), row),
            pl.BlockSpec((_H, 2 * _H), rep),
            pl.BlockSpec((_H, 2 * _H), rep),
            pl.BlockSpec((_H, _H), rep),
            pl.BlockSpec((1, _H), rep),
            pl.BlockSpec((1, _H), rep),
            pl.BlockSpec((1, _H), rep),
        ],
        out_specs=pl.BlockSpec((_BM, out_w), row),
        out_shape=jax.ShapeDtypeStruct(
            (_M, out_w), jnp.int32 if with_mu else jnp.float32),
    )(sums, lfzh, Wz_w, Wh_w, Ur_w, Wz_b2, Wh_b2, Ur_b2)


def _sc_body(mcat_hbm, r1_hbm, idx_hbm, out_hbm,
             idx_all, rows0, rows1, r1b0, r1b1, sums0, sums1,
             g0, g1, r0, r1s, s0, s1):
    wid = lax.axis_index("s") * _NC + lax.axis_index("c")
    wbase = wid * _ROWS_W
    pltpu.sync_copy(idx_hbm.at[pl.ds(wbase * _K, _ROWS_W * _K)], idx_all)

    def fetch(c, rows_b, r1_b, gsem, rsem):
        pltpu.async_copy(
            r1_hbm.at[pl.ds((wbase + c * _CH) * _HW, _CH * _HW)], r1_b,
            rsem)
        pltpu.async_copy(
            mcat_hbm.at[idx_all.at[pl.ds(c * _GI, _GI)]], rows_b, gsem)

    def wait_fetch(c, rows_b, r1_b, gsem, rsem):
        pltpu.make_async_copy(
            r1_hbm.at[pl.ds((wbase + c * _CH) * _HW, _CH * _HW)], r1_b,
            rsem).wait()
        pltpu.make_async_copy(
            mcat_hbm.at[idx_all.at[pl.ds(c * _GI, _GI)]], rows_b,
            gsem).wait()

    def compute(c, rows_b, r1_b, sums_b):
        bf = jnp.bfloat16

        def rowfn(i2, carry2):
            for r in range(2):
                i = 2 * i2 + r
                i6 = i * _K
                for g in range(_H // 32):
                    sl = pl.ds(g * 16, 16)
                    slu = pl.ds(_HW + g * 16, 16)
                    r1v = plsc.bitcast(
                        r1_b[pl.ds(i * _HW + g * 16, 16)], bf)
                    mvs = [plsc.bitcast(rows_b[i6 + k, sl], bf)
                           for k in range(_K)]
                    uvs = [plsc.bitcast(rows_b[i6 + k, slu], bf)
                           for k in range(_K)]
                    gs = [mvs[k] / (1.0 + r1v * uvs[k])
                          for k in range(_K)]
                    acc_m = ((mvs[0] + mvs[1]) + (mvs[2] + mvs[3])
                             + (mvs[4] + mvs[5]))
                    acc_g = ((gs[0] + gs[1]) + (gs[2] + gs[3])
                             + (gs[4] + gs[5]))
                    base = i * _H + g * 16
                    sums_b[pl.ds(base, 16)] = plsc.bitcast(acc_m, jnp.int32)
                    sums_b[pl.ds(base + _HW, 16)] = (
                        plsc.bitcast(acc_g, jnp.int32))
            return carry2
        lax.fori_loop(0, _CH // 2, rowfn, 0)

    def store(c, sums_b, ssem):
        pltpu.async_copy(
            sums_b,
            out_hbm.at[pl.ds((wbase + c * _CH) * _H, _CH * _H)],
            ssem)

    def wait_store(c, sums_b, ssem):
        pltpu.make_async_copy(
            sums_b,
            out_hbm.at[pl.ds((wbase + c * _CH) * _H, _CH * _H)],
            ssem).wait()

    fetch(0, rows0, r1b0, g0, r0)
    fetch(1, rows1, r1b1, g1, r1s)

    def pair(t, carry):
        c0 = 2 * t
        c1 = c0 + 1
        wait_fetch(c0, rows0, r1b0, g0, r0)

        @pl.when(t > 0)
        def _():
            wait_store(c0 - 2, sums0, s0)

        compute(c0, rows0, r1b0, sums0)
        store(c0, sums0, s0)

        @pl.when(t < _NCH // 2 - 1)
        def _():
            fetch(c0 + 2, rows0, r1b0, g0, r0)

        wait_fetch(c1, rows1, r1b1, g1, r1s)

        @pl.when(t > 0)
        def _():
            wait_store(c1 - 2, sums1, s1)

        compute(c1, rows1, r1b1, sums1)
        store(c1, sums1, s1)

        @pl.when(t < _NCH // 2 - 1)
        def _():
            fetch(c1 + 2, rows1, r1b1, g1, r1s)

        return carry

    lax.fori_loop(0, _NCH // 2, pair, 0)
    wait_store(_NCH - 2, sums0, s0)
    wait_store(_NCH - 1, sums1, s1)


def _sc_gate(mcat, r1, idx):
    fn = functools.partial(
        pl.kernel,
        mesh=plsc.VectorSubcoreMesh(core_axis_name="c", subcore_axis_name="s"),
        compiler_params=pltpu.CompilerParams(needs_layout_passes=False),
        out_type=jax.ShapeDtypeStruct((_M * _H,), jnp.int32),
        scratch_types=[
            pltpu.VMEM((_ROWS_W * _K,), jnp.int32),
            pltpu.VMEM((_GI, _H), jnp.int32),
            pltpu.VMEM((_GI, _H), jnp.int32),
            pltpu.VMEM((_CH * _HW,), jnp.int32),
            pltpu.VMEM((_CH * _HW,), jnp.int32),
            pltpu.VMEM((_CH * _H,), jnp.int32),
            pltpu.VMEM((_CH * _H,), jnp.int32),
            pltpu.SemaphoreType.DMA,
            pltpu.SemaphoreType.DMA,
            pltpu.SemaphoreType.DMA,
            pltpu.SemaphoreType.DMA,
            pltpu.SemaphoreType.DMA,
            pltpu.SemaphoreType.DMA,
        ],
    )(_sc_body)
    return fn(mcat, r1.reshape(-1), idx).reshape(_M, _H)


def kernel(messages, local_field, mess_graph, Wz_w, Wz_b, Wr_w, Ur_w, Ur_b,
           Wh_w, Wh_b):
    Wz_b2 = Wz_b.reshape(1, _H)
    Wh_b2 = Wh_b.reshape(1, _H)
    Ur_b2 = Ur_b.reshape(1, _H)
    idx = mess_graph.reshape(-1)

    r1, lfzh, mcat = _tc_pre(local_field, messages, Wz_w, Wr_w, Ur_w,
                             Ur_b2, Wh_w)
    for d in range(_DEPTH):
        sums = _sc_gate(mcat, r1, idx)
        with_mu = d < _DEPTH - 1
        mcat = _tc_post(sums, lfzh, Wz_w, Wh_w, Ur_w, Wz_b2, Wh_b2, Ur_b2,
                        with_mu)
    return mcat
